# baseline (speedup 1.0000x reference)
import math

import jax
import jax.numpy as jnp
from jax import lax
from jax.experimental import pallas as pl
from jax.experimental.pallas import tpu as pltpu
from jax.experimental.pallas import tpu_sc as plsc

D_MODEL = 64
SCALE = math.sqrt(D_MODEL)
VOCAB = 1000000

_NC = 2
_NS = 16
_NW = _NC * _NS
_CHUNK = 128
_NBUF = 2


def _make_sc_lookup(n_rows):
    assert n_rows % (_NW * _CHUNK * _NBUF) == 0
    chunks_per_w = n_rows // (_NW * _CHUNK)
    n_super = chunks_per_w // _NBUF
    mesh = plsc.VectorSubcoreMesh(core_axis_name="c", subcore_axis_name="s")

    def body(idx_hbm, t2p, out_hbm, idx_v, gbufs, sbufs, gsems, ssems):
        wid = lax.axis_index("s") * _NC + lax.axis_index("c")
        chunk_base = wid * chunks_per_w
        pltpu.sync_copy(idx_hbm.at[pl.ds(chunk_base, chunks_per_w)], idx_v)

        def start_gather(b, g):
            pltpu.async_copy(t2p.at[idx_v.at[g]], gbufs[b], gsems[b])

        def wait_gather(b, g):
            pltpu.make_async_copy(
                t2p.at[idx_v.at[g]], gbufs[b], gsems[b]
            ).wait()

        def out_slice(g):
            return out_hbm.at[pl.ds((chunk_base + g) * (_CHUNK // 2), _CHUNK // 2)]

        def start_store(b, g):
            pltpu.async_copy(sbufs[b], out_slice(g), ssems[b])

        def wait_store(b, g):
            pltpu.make_async_copy(sbufs[b], out_slice(g), ssems[b]).wait()

        def scale(b, g):
            def scale_row(r, c):
                for d in range(D_MODEL // 16):
                    sl = pl.ds(d * 16, 16)
                    sbufs[b][r, sl] = gbufs[b][2 * r, sl] * SCALE
                    sbufs[b][r, pl.ds(64 + d * 16, 16)] = (
                        gbufs[b][2 * r + 1, sl] * SCALE
                    )
                return c

            lax.fori_loop(0, _CHUNK // 2, scale_row, 0)

        for b in range(_NBUF):
            start_gather(b, b)
        for b in range(_NBUF):
            wait_gather(b, b)
            scale(b, b)
            start_store(b, b)
            start_gather(b, b + _NBUF)

        def super_it(s, carry):
            for b in range(_NBUF):
                g = s * _NBUF + b
                wait_gather(b, g)
                wait_store(b, g - _NBUF)
                scale(b, g)
                start_store(b, g)
                start_gather(b, g + _NBUF)
            return carry

        if n_super > 2:
            lax.fori_loop(1, n_super - 1, super_it, 0)

        for b in range(_NBUF):
            g = (n_super - 1) * _NBUF + b
            wait_gather(b, g)
            wait_store(b, g - _NBUF)
            scale(b, g)
            start_store(b, g)
        for b in range(_NBUF):
            g = (n_super - 1) * _NBUF + b
            wait_store(b, g)

    return pl.kernel(
        body,
        out_type=jax.ShapeDtypeStruct((n_rows // 2, 2 * D_MODEL), jnp.float32),
        mesh=mesh,
        scratch_types=[
            pltpu.VMEM((chunks_per_w, _CHUNK), jnp.int32),
            [pltpu.VMEM((_CHUNK, 2 * D_MODEL), jnp.float32) for _ in range(_NBUF)],
            [pltpu.VMEM((_CHUNK // 2, 2 * D_MODEL), jnp.float32) for _ in range(_NBUF)],
            [pltpu.SemaphoreType.DMA for _ in range(_NBUF)],
            [pltpu.SemaphoreType.DMA for _ in range(_NBUF)],
        ],
    )


def kernel(x, embedding):
    n_rows = x.size
    idx = x.reshape(n_rows // _CHUNK, _CHUNK).astype(jnp.int32)
    t2d = jnp.concatenate([embedding, embedding], axis=1)
    out2 = _make_sc_lookup(n_rows)(idx, t2d)
    return out2.reshape(x.shape + (D_MODEL,))


# speedup vs baseline: 1.4202x; 1.4202x over previous
import math

import jax
import jax.numpy as jnp
from jax import lax
from jax.experimental import pallas as pl
from jax.experimental.pallas import tpu as pltpu
from jax.experimental.pallas import tpu_sc as plsc

D_MODEL = 64
SCALE = math.sqrt(D_MODEL)
VOCAB = 1000000

_NC = 2
_NS = 16
_NW = _NC * _NS
_CHUNK = 128
_NBUF = 2


def _make_sc_lookup(n_rows):
    assert n_rows % (_NW * _CHUNK * _NBUF) == 0
    chunks_per_w = n_rows // (_NW * _CHUNK)
    n_super = chunks_per_w // _NBUF
    mesh = plsc.VectorSubcoreMesh(core_axis_name="c", subcore_axis_name="s")

    def body(idx_hbm, t2p, out_hbm, idx_v, gbufs, sbufs, gsems, ssems):
        wid = lax.axis_index("s") * _NC + lax.axis_index("c")
        chunk_base = wid * chunks_per_w
        pltpu.sync_copy(idx_hbm.at[pl.ds(chunk_base, chunks_per_w)], idx_v)

        def start_gather(b, g):
            pltpu.async_copy(t2p.at[idx_v.at[g]], gbufs[b], gsems[b])

        def wait_gather(b, g):
            pltpu.make_async_copy(
                t2p.at[idx_v.at[g]], gbufs[b], gsems[b]
            ).wait()

        def out_slice(g):
            return out_hbm.at[pl.ds((chunk_base + g) * _CHUNK, _CHUNK)]

        def start_store(b, g):
            pltpu.async_copy(sbufs[b], out_slice(g), ssems[b])

        def wait_store(b, g):
            pltpu.make_async_copy(sbufs[b], out_slice(g), ssems[b]).wait()

        def scale(b, g):
            def scale_row(r, c):
                for d in range(D_MODEL // 16):
                    sl = pl.ds(d * 16, 16)
                    sbufs[b][r, sl] = gbufs[b][r, sl] * SCALE
                return c

            lax.fori_loop(0, _CHUNK, scale_row, 0)

        for b in range(_NBUF):
            start_gather(b, b)
        for b in range(_NBUF):
            wait_gather(b, b)
            scale(b, b)
            start_store(b, b)
            start_gather(b, b + _NBUF)

        def super_it(s, carry):
            for b in range(_NBUF):
                g = s * _NBUF + b
                wait_gather(b, g)
                wait_store(b, g - _NBUF)
                scale(b, g)
                start_store(b, g)
                start_gather(b, g + _NBUF)
            return carry

        if n_super > 2:
            lax.fori_loop(1, n_super - 1, super_it, 0)

        for b in range(_NBUF):
            g = (n_super - 1) * _NBUF + b
            wait_gather(b, g)
            wait_store(b, g - _NBUF)
            scale(b, g)
            start_store(b, g)
        for b in range(_NBUF):
            g = (n_super - 1) * _NBUF + b
            wait_store(b, g)

    return pl.kernel(
        body,
        out_type=jax.ShapeDtypeStruct((n_rows, D_MODEL), jnp.float32),
        mesh=mesh,
        scratch_types=[
            pltpu.VMEM((chunks_per_w, _CHUNK), jnp.int32),
            [pltpu.VMEM((_CHUNK, 2 * D_MODEL), jnp.float32) for _ in range(_NBUF)],
            [pltpu.VMEM((_CHUNK, D_MODEL), jnp.float32) for _ in range(_NBUF)],
            [pltpu.SemaphoreType.DMA for _ in range(_NBUF)],
            [pltpu.SemaphoreType.DMA for _ in range(_NBUF)],
        ],
    )


def kernel(x, embedding):
    n_rows = x.size
    idx = x.reshape(n_rows // _CHUNK, _CHUNK).astype(jnp.int32)
    t2d = jnp.concatenate([embedding, embedding], axis=1)
    out = _make_sc_lookup(n_rows)(idx, t2d)
    return out.reshape(x.shape + (D_MODEL,))
